# Initial kernel scaffold; baseline (speedup 1.0000x reference)
import jax
import jax.numpy as jnp
from jax.experimental import pallas as pl


def kernel(x, edge_index, params):
    raise NotImplementedError("write your pallas kernel here")


# speedup vs baseline: 32.7935x; 32.7935x over previous
import functools

import jax
import jax.numpy as jnp
from jax.experimental import pallas as pl
from jax.experimental.pallas import tpu as pltpu

N = 307
NP = 384
T = 12
CC = 64
CT = 64
KCH = 3
PRED = 12
NEG = -1e30

_f32 = jnp.float32


def _block_compute(x, u1, U2, u3, be, Ve, W1, W2, W3, bs, Vs,
                   chebw, chebb, tw, tb, rw, rb, lng, lnb, Cm, FP):
    X_t = [x[t * NP:(t + 1) * NP, :] for t in range(T)]

    a1_rows = [jnp.dot(u1, X_t[t], preferred_element_type=_f32)
               for t in range(T)]
    A1 = jnp.concatenate(a1_rows, axis=0)
    LHS = jax.lax.dot_general(A1, U2, (((1,), (0,)), ((), ())),
                              preferred_element_type=_f32)
    rhs_cols = [jax.lax.dot_general(X_t[t], u3, (((1,), (1,)), ((), ())),
                                    preferred_element_type=_f32)
                for t in range(T)]
    RHS = jnp.concatenate(rhs_cols, axis=1)
    E = jnp.dot(LHS, RHS, preferred_element_type=_f32)
    E2 = jnp.dot(Ve, jax.nn.sigmoid(E + be),
                 preferred_element_type=_f32)
    Em = jnp.max(E2, axis=0, keepdims=True)
    Ee = jnp.exp(E2 - Em)
    Et = Ee / jnp.sum(Ee, axis=0, keepdims=True)

    a = jax.lax.dot_general(Et, W1, (((1,), (1,)), ((), ())),
                            preferred_element_type=_f32)
    B1 = a[0:1, 0:1] * X_t[0]
    for s in range(1, T):
        B1 = B1 + a[s:s + 1, 0:1] * X_t[s]
    LHS2 = jax.lax.dot_general(B1, W2, (((1,), (0,)), ((), ())),
                               preferred_element_type=_f32)
    d_cols = [jax.lax.dot_general(X_t[t], W3, (((1,), (1,)), ((), ())),
                                  preferred_element_type=_f32)
              for t in range(T)]
    D = jnp.concatenate(d_cols, axis=1)
    C1 = jnp.dot(D, Et, preferred_element_type=_f32)
    S = jax.lax.dot_general(LHS2, C1, (((1,), (1,)), ((), ())),
                            preferred_element_type=_f32)
    S2 = jnp.dot(Vs, jax.nn.sigmoid(S + bs),
                 preferred_element_type=_f32)
    rowid = jax.lax.broadcasted_iota(jnp.int32, (NP, NP), 0)
    S2 = jnp.where(rowid < N, S2, NEG)
    Sm_ = jnp.max(S2, axis=0, keepdims=True)
    Se = jnp.exp(S2 - Sm_)
    Sm = Se / jnp.sum(Se, axis=0, keepdims=True)

    colid = jax.lax.broadcasted_iota(jnp.int32, (NP, NP), 1)
    eye = rowid == colid
    dcol = jnp.sum(jnp.where(eye, Sm, 0.0), axis=1, keepdims=True)
    CS = Cm * Sm

    H0 = jnp.concatenate([dcol * X_t[t] for t in range(T)], axis=1)
    H1 = jax.lax.dot_general(CS, H0, (((0,), (0,)), ((), ())),
                             preferred_element_type=_f32)
    H2 = 2.0 * jax.lax.dot_general(CS, H1, (((0,), (0,)), ((), ())),
                                   preferred_element_type=_f32) - H0
    Wk = [chebw[k * FP:(k + 1) * FP, :] for k in range(KCH)]
    Xhat_t = []
    for t in range(T):
        sl = slice(t * FP, (t + 1) * FP)
        o = (jnp.dot(H0[:, sl], Wk[0], preferred_element_type=_f32)
             + jnp.dot(H1[:, sl], Wk[1], preferred_element_type=_f32)
             + jnp.dot(H2[:, sl], Wk[2], preferred_element_type=_f32)
             + chebb)
        Xhat_t.append(jnp.maximum(o, 0.0))

    TW = [tw[w * CC:(w + 1) * CC, :] for w in range(3)]
    out_rows = []
    for t in range(T):
        acc = tb + jnp.dot(X_t[t], rw, preferred_element_type=_f32) + rb
        for w in range(3):
            tt = t + w - 1
            if 0 <= tt < T:
                acc = acc + jnp.dot(Xhat_t[tt], TW[w],
                                    preferred_element_type=_f32)
        Z = jnp.maximum(acc, 0.0)
        mu = jnp.mean(Z, axis=1, keepdims=True)
        var = jnp.mean(Z * Z, axis=1, keepdims=True) - mu * mu
        ZN = (Z - mu) * jax.lax.rsqrt(var + 1e-5) * lng + lnb
        out_rows.append(ZN)
    return jnp.concatenate(out_rows, axis=0)


def _block_kernel(FP, x_ref, u1, U2, u3, be, Ve, W1, W2, W3, bs, Vs,
                  chebw, chebb, tw, tb, rw, rb, lng, lnb, Cm, o_ref):
    out = _block_compute(x_ref[0], u1[...], U2[...], u3[...], be[...],
                         Ve[...], W1[...], W2[...], W3[...], bs[...],
                         Vs[...], chebw[...], chebb[...], tw[...], tb[...],
                         rw[...], rb[...], lng[...], lnb[...], Cm[...], FP)
    o_ref[0] = out


def _full(shape):
    nd = len(shape)
    return pl.BlockSpec(shape, lambda b: (0,) * nd)


def _run_block(x, wlist, FP, B):
    in_specs = [pl.BlockSpec((1, T * NP, FP), lambda b: (b, 0, 0))]
    in_specs += [_full(w.shape) for w in wlist]
    return pl.pallas_call(
        functools.partial(_block_kernel, FP),
        grid=(B,),
        in_specs=in_specs,
        out_specs=pl.BlockSpec((1, T * NP, CT), lambda b: (b, 0, 0)),
        out_shape=jax.ShapeDtypeStruct((B, T * NP, CT), _f32),
        compiler_params=pltpu.CompilerParams(
            dimension_semantics=("parallel",)),
    )(x, *wlist)


def _final_kernel(with_affine, x_ref, fw, fb, lw, o_ref):
    x = x_ref[0]
    acc = fb[...]
    for t in range(T):
        acc = acc + jnp.dot(x[t * NP:(t + 1) * NP, :],
                            fw[t * CC:(t + 1) * CC, :],
                            preferred_element_type=_f32)
    acc = jnp.maximum(acc, 0.0)
    if with_affine:
        acc = acc * lw[0:1, 0:1] + lw[1:2, 0:1]
    o_ref[0] = acc


def _run_final(x, fw, fb, lw, with_affine, B):
    return pl.pallas_call(
        functools.partial(_final_kernel, with_affine),
        grid=(B,),
        in_specs=[pl.BlockSpec((1, T * NP, CT), lambda b: (b, 0, 0)),
                  _full(fw.shape), _full(fb.shape), _full(lw.shape)],
        out_specs=pl.BlockSpec((1, NP, PRED), lambda b: (b, 0, 0)),
        out_shape=jax.ShapeDtypeStruct((B, NP, PRED), _f32),
        compiler_params=pltpu.CompilerParams(
            dimension_semantics=("parallel",)),
    )(x, fw, fb, lw)


def _pad2(a, r, c):
    return jnp.pad(a, ((0, r - a.shape[0]), (0, c - a.shape[1])))


def _prep_block_weights(p, F, FP):
    u1 = _pad2(p['U1'][None, :], 1, NP)
    U2 = _pad2(p['U2'], FP, NP)
    u3 = _pad2(p['U3'][None, :], 1, FP)
    be = p['be'][0]
    Ve = p['Ve']
    W1 = p['W1'][None, :]
    W2 = _pad2(p['W2'], FP, T)
    W3 = _pad2(p['W3'][None, :], 1, FP)
    bs = _pad2(p['bs'][0], NP, NP)
    Vs = _pad2(p['Vs'], NP, NP)
    chebw = jnp.concatenate(
        [_pad2(p['cheb_w'][k], FP, CC) for k in range(KCH)], axis=0)
    chebb = p['cheb_b'][None, :]
    tw = jnp.concatenate(
        [jnp.transpose(p['time_w'][:, :, 0, w]) for w in range(3)], axis=0)
    tb = p['time_b'][None, :]
    rw = _pad2(jnp.transpose(p['res_w'][:, :, 0, 0]), FP, CT)
    rb = p['res_b'][None, :]
    lng = p['ln_g'][None, :]
    lnb = p['ln_b'][None, :]
    return [u1, U2, u3, be, Ve, W1, W2, W3, bs, Vs, chebw, chebb,
            tw, tb, rw, rb, lng, lnb]


def _edge_matrix(edge_index):
    row, col = edge_index[0], edge_index[1]
    mask = (row != col).astype(_f32)
    deg = jnp.zeros((N,), _f32).at[row].add(mask)
    dinv = jnp.where(deg > 0, jax.lax.rsqrt(jnp.where(deg > 0, deg, 1.0)),
                     0.0)
    normv = -dinv[row] * dinv[col] * mask
    Cm = jnp.zeros((NP, NP), _f32).at[row, col].add(normv)
    return Cm


def _astgcn(x, params, Cm, B):
    w0 = _prep_block_weights(params['blocks'][0], 1, 8) + [Cm]
    h = _run_block(x, w0, 8, B)
    w1 = _prep_block_weights(params['blocks'][1], CT, CT) + [Cm]
    h = _run_block(h, w1, CT, B)
    fw = jnp.concatenate(
        [jnp.transpose(params['final_w'][:, t, 0, :]) for t in range(T)],
        axis=0)
    fb = params['final_b'][None, :]
    return h, fw, fb


def kernel(x, edge_index, params):
    B = x.shape[0]
    Cm = _edge_matrix(edge_index)

    x1 = jnp.transpose(x, (0, 3, 1, 2))
    x1 = jnp.pad(x1, ((0, 0), (0, 0), (0, NP - N), (0, 7)))
    x1 = x1.reshape(B, T * NP, 8)
    h, fw, fb = _astgcn(x1, params['astgcn1'], Cm, B)
    lw_dummy = jnp.zeros((2, 1), _f32)
    h = _run_final(h, fw, fb, lw_dummy, False, B)

    x2 = jnp.transpose(h, (0, 2, 1))[..., None]
    x2 = jnp.pad(x2, ((0, 0), (0, 0), (0, 0), (0, 7)))
    x2 = x2.reshape(B, T * NP, 8)
    h2, fw2, fb2 = _astgcn(x2, params['astgcn2'], Cm, B)
    lw = jnp.concatenate([params['lin_w'][0:1, 0:1],
                          params['lin_b'][None, 0:1]], axis=0)
    y = _run_final(h2, fw2, fb2, lw, True, B)

    return y[:, :N, :, None]


# speedup vs baseline: 36.9312x; 1.1262x over previous
import functools

import jax
import jax.numpy as jnp
from jax.experimental import pallas as pl
from jax.experimental.pallas import tpu as pltpu
from jax.experimental.pallas import tpu_sc as plsc

N = 307
NP = 384
T = 12
CC = 64
CT = 64
KCH = 3
PRED = 12
NEG = -1e30

_f32 = jnp.float32


def _block_compute(x, u1, U2, u3, be, Ve, W1, W2, W3, bs, Vs,
                   chebw, chebb, tw, tb, rw, rb, lng, lnb, Cm, FP):
    X_t = [x[t * NP:(t + 1) * NP, :] for t in range(T)]

    a1_rows = [jnp.dot(u1, X_t[t], preferred_element_type=_f32)
               for t in range(T)]
    A1 = jnp.concatenate(a1_rows, axis=0)
    LHS = jax.lax.dot_general(A1, U2, (((1,), (0,)), ((), ())),
                              preferred_element_type=_f32)
    rhs_cols = [jax.lax.dot_general(X_t[t], u3, (((1,), (1,)), ((), ())),
                                    preferred_element_type=_f32)
                for t in range(T)]
    RHS = jnp.concatenate(rhs_cols, axis=1)
    E = jnp.dot(LHS, RHS, preferred_element_type=_f32)
    E2 = jnp.dot(Ve, jax.nn.sigmoid(E + be),
                 preferred_element_type=_f32)
    Em = jnp.max(E2, axis=0, keepdims=True)
    Ee = jnp.exp(E2 - Em)
    Et = Ee / jnp.sum(Ee, axis=0, keepdims=True)

    a = jax.lax.dot_general(Et, W1, (((1,), (1,)), ((), ())),
                            preferred_element_type=_f32)
    B1 = a[0:1, 0:1] * X_t[0]
    for s in range(1, T):
        B1 = B1 + a[s:s + 1, 0:1] * X_t[s]
    LHS2 = jax.lax.dot_general(B1, W2, (((1,), (0,)), ((), ())),
                               preferred_element_type=_f32)
    d_cols = [jax.lax.dot_general(X_t[t], W3, (((1,), (1,)), ((), ())),
                                  preferred_element_type=_f32)
              for t in range(T)]
    D = jnp.concatenate(d_cols, axis=1)
    C1 = jnp.dot(D, Et, preferred_element_type=_f32)
    S = jax.lax.dot_general(LHS2, C1, (((1,), (1,)), ((), ())),
                            preferred_element_type=_f32)
    S2 = jnp.dot(Vs, jax.nn.sigmoid(S + bs),
                 preferred_element_type=_f32)
    rowid = jax.lax.broadcasted_iota(jnp.int32, (NP, NP), 0)
    S2 = jnp.where(rowid < N, S2, NEG)
    Sm_ = jnp.max(S2, axis=0, keepdims=True)
    Se = jnp.exp(S2 - Sm_)
    Sm = Se / jnp.sum(Se, axis=0, keepdims=True)

    colid = jax.lax.broadcasted_iota(jnp.int32, (NP, NP), 1)
    eye = rowid == colid
    dcol = jnp.sum(jnp.where(eye, Sm, 0.0), axis=1, keepdims=True)
    CS = Cm * Sm

    H0 = jnp.concatenate([dcol * X_t[t] for t in range(T)], axis=1)
    H1 = jax.lax.dot_general(CS, H0, (((0,), (0,)), ((), ())),
                             preferred_element_type=_f32)
    H2 = 2.0 * jax.lax.dot_general(CS, H1, (((0,), (0,)), ((), ())),
                                   preferred_element_type=_f32) - H0
    Wk = [chebw[k * FP:(k + 1) * FP, :] for k in range(KCH)]
    Xhat_t = []
    for t in range(T):
        sl = slice(t * FP, (t + 1) * FP)
        o = (jnp.dot(H0[:, sl], Wk[0], preferred_element_type=_f32)
             + jnp.dot(H1[:, sl], Wk[1], preferred_element_type=_f32)
             + jnp.dot(H2[:, sl], Wk[2], preferred_element_type=_f32)
             + chebb)
        Xhat_t.append(jnp.maximum(o, 0.0))

    TW = [tw[w * CC:(w + 1) * CC, :] for w in range(3)]
    out_rows = []
    for t in range(T):
        acc = tb + jnp.dot(X_t[t], rw, preferred_element_type=_f32) + rb
        for w in range(3):
            tt = t + w - 1
            if 0 <= tt < T:
                acc = acc + jnp.dot(Xhat_t[tt], TW[w],
                                    preferred_element_type=_f32)
        Z = jnp.maximum(acc, 0.0)
        mu = jnp.mean(Z, axis=1, keepdims=True)
        var = jnp.mean(Z * Z, axis=1, keepdims=True) - mu * mu
        ZN = (Z - mu) * jax.lax.rsqrt(var + 1e-5) * lng + lnb
        out_rows.append(ZN)
    return jnp.concatenate(out_rows, axis=0)


def _block_kernel(FP, x_ref, u1, U2, u3, be, Ve, W1, W2, W3, bs, Vs,
                  chebw, chebb, tw, tb, rw, rb, lng, lnb, Cm, o_ref):
    out = _block_compute(x_ref[0], u1[...], U2[...], u3[...], be[...],
                         Ve[...], W1[...], W2[...], W3[...], bs[...],
                         Vs[...], chebw[...], chebb[...], tw[...], tb[...],
                         rw[...], rb[...], lng[...], lnb[...], Cm[...], FP)
    o_ref[0] = out


def _full(shape):
    nd = len(shape)
    return pl.BlockSpec(shape, lambda b: (0,) * nd)


def _run_block(x, wlist, FP, B):
    in_specs = [pl.BlockSpec((1, T * NP, FP), lambda b: (b, 0, 0))]
    in_specs += [_full(w.shape) for w in wlist]
    return pl.pallas_call(
        functools.partial(_block_kernel, FP),
        grid=(B,),
        in_specs=in_specs,
        out_specs=pl.BlockSpec((1, T * NP, CT), lambda b: (b, 0, 0)),
        out_shape=jax.ShapeDtypeStruct((B, T * NP, CT), _f32),
        compiler_params=pltpu.CompilerParams(
            dimension_semantics=("parallel",)),
    )(x, *wlist)


def _final_kernel(with_affine, x_ref, fw, fb, lw, o_ref):
    x = x_ref[0]
    acc = fb[...]
    for t in range(T):
        acc = acc + jnp.dot(x[t * NP:(t + 1) * NP, :],
                            fw[t * CC:(t + 1) * CC, :],
                            preferred_element_type=_f32)
    acc = jnp.maximum(acc, 0.0)
    if with_affine:
        acc = acc * lw[0:1, 0:1] + lw[1:2, 0:1]
    o_ref[0] = acc


def _run_final(x, fw, fb, lw, with_affine, B):
    return pl.pallas_call(
        functools.partial(_final_kernel, with_affine),
        grid=(B,),
        in_specs=[pl.BlockSpec((1, T * NP, CT), lambda b: (b, 0, 0)),
                  _full(fw.shape), _full(fb.shape), _full(lw.shape)],
        out_specs=pl.BlockSpec((1, NP, PRED), lambda b: (b, 0, 0)),
        out_shape=jax.ShapeDtypeStruct((B, NP, PRED), _f32),
        compiler_params=pltpu.CompilerParams(
            dimension_semantics=("parallel",)),
    )(x, fw, fb, lw)


def _pad2(a, r, c):
    return jnp.pad(a, ((0, r - a.shape[0]), (0, c - a.shape[1])))


def _prep_block_weights(p, F, FP):
    u1 = _pad2(p['U1'][None, :], 1, NP)
    U2 = _pad2(p['U2'], FP, NP)
    u3 = _pad2(p['U3'][None, :], 1, FP)
    be = p['be'][0]
    Ve = p['Ve']
    W1 = p['W1'][None, :]
    W2 = _pad2(p['W2'], FP, T)
    W3 = _pad2(p['W3'][None, :], 1, FP)
    bs = _pad2(p['bs'][0], NP, NP)
    Vs = _pad2(p['Vs'], NP, NP)
    chebw = jnp.concatenate(
        [_pad2(p['cheb_w'][k], FP, CC) for k in range(KCH)], axis=0)
    chebb = p['cheb_b'][None, :]
    tw = jnp.concatenate(
        [jnp.transpose(p['time_w'][:, :, 0, w]) for w in range(3)], axis=0)
    tb = p['time_b'][None, :]
    rw = _pad2(jnp.transpose(p['res_w'][:, :, 0, 0]), FP, CT)
    rb = p['res_b'][None, :]
    lng = p['ln_g'][None, :]
    lnb = p['ln_b'][None, :]
    return [u1, U2, u3, be, Ve, W1, W2, W3, bs, Vs, chebw, chebb,
            tw, tb, rw, rb, lng, lnb]


NE = 4912
NCHUNK = NE // 16
NDEG = 320
CFLAT = N * NP


def _edge_sc_body(edges_hbm, out_hbm, ev, deg, dinv, cflat):
    wid = jax.lax.axis_index("c") * 16 + jax.lax.axis_index("s")

    @pl.when(wid == 0)
    def _():
        pltpu.sync_copy(edges_hbm, ev)
        for i in range(NDEG // 16):
            deg[pl.ds(i * 16, 16)] = jnp.zeros((16,), _f32)

        def deg_body(i, carry):
            r = ev[pl.ds(i * 16, 16)]
            c = ev[pl.ds(NE + i * 16, 16)]
            mf = jnp.where(r != c, 1.0, 0.0).astype(_f32)
            plsc.addupdate_scatter(deg, [r], mf)
            return carry
        jax.lax.fori_loop(0, NCHUNK, deg_body, 0)

        for i in range(NDEG // 16):
            d = deg[pl.ds(i * 16, 16)]
            bits = plsc.bitcast(d, jnp.int32)
            y = plsc.bitcast(jnp.int32(0x5F3759DF) - (bits >> 1), _f32)
            for _ in range(4):
                y = y * (1.5 - 0.5 * d * y * y)
            dinv[pl.ds(i * 16, 16)] = jnp.where(d > 0.5, y, 0.0)

        def zero_body(i, carry):
            cflat[pl.ds(i * 16, 16)] = jnp.zeros((16,), _f32)
            return carry
        jax.lax.fori_loop(0, CFLAT // 16, zero_body, 0)

        def c_body(i, carry):
            r = ev[pl.ds(i * 16, 16)]
            c = ev[pl.ds(NE + i * 16, 16)]
            mf = jnp.where(r != c, -1.0, 0.0).astype(_f32)
            dr = plsc.load_gather(dinv, [r])
            dc = plsc.load_gather(dinv, [c])
            plsc.addupdate_scatter(cflat, [r * NP + c], dr * dc * mf)
            return carry
        jax.lax.fori_loop(0, NCHUNK, c_body, 0)

        pltpu.sync_copy(cflat, out_hbm)


_edge_sc = functools.partial(
    pl.kernel,
    out_type=jax.ShapeDtypeStruct((CFLAT,), _f32),
    mesh=plsc.VectorSubcoreMesh(core_axis_name="c", subcore_axis_name="s"),
    compiler_params=pltpu.CompilerParams(needs_layout_passes=False),
    scratch_types=[pltpu.VMEM((2 * NE,), jnp.int32),
                   pltpu.VMEM((NDEG,), _f32),
                   pltpu.VMEM((NDEG,), _f32),
                   pltpu.VMEM((CFLAT,), _f32)],
)(_edge_sc_body)


def _edge_matrix(edge_index):
    cm_flat = _edge_sc(edge_index.reshape(2 * NE))
    return jnp.pad(cm_flat.reshape(N, NP), ((0, NP - N), (0, 0)))


def _astgcn(x, params, Cm, B):
    w0 = _prep_block_weights(params['blocks'][0], 1, 8) + [Cm]
    h = _run_block(x, w0, 8, B)
    w1 = _prep_block_weights(params['blocks'][1], CT, CT) + [Cm]
    h = _run_block(h, w1, CT, B)
    fw = jnp.concatenate(
        [jnp.transpose(params['final_w'][:, t, 0, :]) for t in range(T)],
        axis=0)
    fb = params['final_b'][None, :]
    return h, fw, fb


def kernel(x, edge_index, params):
    B = x.shape[0]
    Cm = _edge_matrix(edge_index)

    x1 = jnp.transpose(x, (0, 3, 1, 2))
    x1 = jnp.pad(x1, ((0, 0), (0, 0), (0, NP - N), (0, 7)))
    x1 = x1.reshape(B, T * NP, 8)
    h, fw, fb = _astgcn(x1, params['astgcn1'], Cm, B)
    lw_dummy = jnp.zeros((2, 1), _f32)
    h = _run_final(h, fw, fb, lw_dummy, False, B)

    x2 = jnp.transpose(h, (0, 2, 1))[..., None]
    x2 = jnp.pad(x2, ((0, 0), (0, 0), (0, 0), (0, 7)))
    x2 = x2.reshape(B, T * NP, 8)
    h2, fw2, fb2 = _astgcn(x2, params['astgcn2'], Cm, B)
    lw = jnp.concatenate([params['lin_w'][0:1, 0:1],
                          params['lin_b'][None, 0:1]], axis=0)
    y = _run_final(h2, fw2, fb2, lw, True, B)

    return y[:, :N, :, None]
